# Initial kernel scaffold; baseline (speedup 1.0000x reference)
import jax
import jax.numpy as jnp
from jax.experimental import pallas as pl


def kernel(grid, emb, W, b):
    raise NotImplementedError("write your pallas kernel here")


# speedup vs baseline: 100.6446x; 100.6446x over previous
import functools

import jax
import jax.numpy as jnp
from jax import lax
from jax.experimental import pallas as pl
from jax.experimental.pallas import tpu as pltpu
from jax.experimental.pallas import tpu_sc as plsc

GRID_H = 1024
GRID_W = 1024
NUM_COLORS = 10
D_MODEL = 128

NC = 2
NS = 16
NW = NC * NS
NPT = (GRID_H * GRID_W) // NW
LANES = 16
CSLOTS = 16
HIST = CSLOTS * LANES
UNROLL = 8


def _sc_hist_body(grid_hbm, out_hbm, buf_v, hist_v):
    wid = lax.axis_index("c") * NS + lax.axis_index("s")
    base = wid * NPT

    zero16 = jnp.zeros((LANES,), jnp.float32)
    for i in range(HIST // LANES):
        hist_v[pl.ds(i * LANES, LANES)] = zero16

    pltpu.sync_copy(grid_hbm.at[pl.ds(base, NPT)], buf_v)

    lane = lax.iota(jnp.int32, LANES)
    ones = jnp.ones((LANES,), jnp.float32)

    def body(i, carry):
        b0 = i * (LANES * UNROLL)
        for u in range(UNROLL):
            v = buf_v[pl.ds(b0 + u * LANES, LANES)]
            idx = v * LANES + lane
            plsc.addupdate_scatter(hist_v, [idx], ones)
        return carry

    lax.fori_loop(0, NPT // (LANES * UNROLL), body, 0)

    pltpu.sync_copy(hist_v, out_hbm.at[wid])


@functools.cache
def _sc_hist():
    return functools.partial(
        pl.kernel,
        mesh=plsc.VectorSubcoreMesh(core_axis_name="c", subcore_axis_name="s"),
        out_type=jax.ShapeDtypeStruct((NW, HIST), jnp.float32),
        scratch_types=[
            pltpu.VMEM((NPT,), jnp.int32),
            pltpu.VMEM((HIST,), jnp.float32),
        ],
        compiler_params=pltpu.CompilerParams(needs_layout_passes=False),
    )(_sc_hist_body)


def _proj_body(h_ref, embB_ref, W_ref, b_ref, o_ref):
    hs = jnp.sum(h_ref[:], axis=0, keepdims=True)
    x = jnp.dot(hs, embB_ref[:], preferred_element_type=jnp.float32)
    x = x * (1.0 / (GRID_H * GRID_W))
    y = lax.dot_general(x, W_ref[:], (((1,), (1,)), ((), ())),
                        preferred_element_type=jnp.float32)
    o_ref[:] = y + b_ref[:]


@jax.jit
def kernel(grid, emb, W, b):
    gflat = grid.reshape(-1).astype(jnp.int32)
    hist = _sc_hist()(gflat)

    embp = jnp.zeros((CSLOTS, D_MODEL), jnp.float32).at[:NUM_COLORS].set(emb)
    embB = jnp.repeat(embp, LANES, axis=0)

    out = pl.pallas_call(
        _proj_body,
        out_shape=jax.ShapeDtypeStruct((1, D_MODEL), jnp.float32),
    )(hist, embB, W, b.reshape(1, D_MODEL))
    return out.reshape(D_MODEL)


# speedup vs baseline: 157.2909x; 1.5628x over previous
import functools

import jax
import jax.numpy as jnp
from jax import lax
from jax.experimental import pallas as pl
from jax.experimental.pallas import tpu as pltpu
from jax.experimental.pallas import tpu_sc as plsc

GRID_H = 1024
GRID_W = 1024
NUM_COLORS = 10
D_MODEL = 128

NC = 2
NS = 16
NW = NC * NS
NPT = (GRID_H * GRID_W) // NW
LANES = 16
CSLOTS = 16
HIST = CSLOTS * LANES
NREG = 8
HISTT = NREG * HIST
UNROLL = 8


ROWS_PT = GRID_H // NW
NCH = 4
CROWS = ROWS_PT // NCH
GPC = CROWS * GRID_W // LANES


def _sc_hist_body(grid_hbm, out_hbm, buf_v, hist_v, sem0, sem1):
    wid = lax.axis_index("c") * NS + lax.axis_index("s")
    rbase = wid * ROWS_PT
    sems = (sem0, sem1)

    copies = [None] * NCH
    copies[0] = pltpu.async_copy(
        grid_hbm.at[pl.ds(rbase, CROWS)], buf_v.at[pl.ds(0, CROWS)], sem0)

    zero16 = jnp.zeros((LANES,), jnp.float32)
    for i in range(HISTT // LANES):
        hist_v[pl.ds(i * LANES, LANES)] = zero16

    lane = lax.iota(jnp.int32, LANES)
    ones = jnp.ones((LANES,), jnp.float32)

    for k in range(NCH):
        if k + 1 < NCH:
            copies[k + 1] = pltpu.async_copy(
                grid_hbm.at[pl.ds(rbase + (k + 1) * CROWS, CROWS)],
                buf_v.at[pl.ds(((k + 1) % 2) * CROWS, CROWS)],
                sems[(k + 1) % 2])
        copies[k].wait()
        row0 = (k % 2) * CROWS

        @plsc.parallel_loop(0, GPC, unroll=UNROLL)
        def _(i):
            v = buf_v[row0 + (i >> 6), pl.ds((i & 63) * LANES, LANES)]
            region = (i & (NREG - 1)) << 8
            idx = v * LANES + lane + region
            plsc.addupdate_scatter(hist_v, [idx], ones)

    pltpu.sync_copy(hist_v, out_hbm.at[wid])


@functools.cache
def _sc_hist():
    return functools.partial(
        pl.kernel,
        mesh=plsc.VectorSubcoreMesh(core_axis_name="c", subcore_axis_name="s"),
        out_type=jax.ShapeDtypeStruct((NW, HISTT), jnp.float32),
        scratch_types=[
            pltpu.VMEM((2 * CROWS, GRID_W), jnp.int32),
            pltpu.VMEM((HISTT,), jnp.float32),
            pltpu.SemaphoreType.DMA,
            pltpu.SemaphoreType.DMA,
        ],
        compiler_params=pltpu.CompilerParams(needs_layout_passes=False),
    )(_sc_hist_body)


def _proj_body(h_ref, emb_ref, W_ref, b_ref, o_ref):
    hs = jnp.sum(h_ref[:], axis=0, keepdims=True)
    k_idx = lax.broadcasted_iota(jnp.int32, (HISTT, CSLOTS), 0)
    v_idx = lax.broadcasted_iota(jnp.int32, (HISTT, CSLOTS), 1)
    sel = jnp.where(((k_idx >> 4) & (CSLOTS - 1)) == v_idx, 1.0, 0.0)
    c16 = jnp.dot(hs, sel, preferred_element_type=jnp.float32)
    x = jnp.dot(c16[:, :NUM_COLORS], emb_ref[:],
                preferred_element_type=jnp.float32)
    x = x * (1.0 / (GRID_H * GRID_W))
    y = lax.dot_general(x, W_ref[:], (((1,), (1,)), ((), ())),
                        preferred_element_type=jnp.float32)
    o_ref[:] = y + b_ref[:]


@jax.jit
def kernel(grid, emb, W, b):
    hist = _sc_hist()(grid.astype(jnp.int32))

    out = pl.pallas_call(
        _proj_body,
        out_shape=jax.ShapeDtypeStruct((1, D_MODEL), jnp.float32),
    )(hist, emb, W, b)
    return out.reshape(D_MODEL)
